# baseline (speedup 1.0000x reference)
import functools
import math

import jax
import jax.numpy as jnp
from jax import lax
from jax.experimental import pallas as pl
from jax.experimental.pallas import tpu as pltpu
from jax.experimental.pallas import tpu_sc as plsc

_NUM_CORES = 2
_NUM_SUBCORES = 16
_NW = _NUM_CORES * _NUM_SUBCORES
_LANES = 16
_CHUNK = 128
_NG = 4
_NO = 2


def _make_lookup(d_model, n_idx, scale):
    per_w = n_idx // _NW
    n_chunk = per_w // _CHUNK
    n_outer = n_chunk // _NG
    n_vec = d_model // _LANES
    mesh = plsc.VectorSubcoreMesh(core_axis_name="c", subcore_axis_name="s")

    @functools.partial(
        pl.kernel,
        mesh=mesh,
        out_type=jax.ShapeDtypeStruct((n_idx, d_model), jnp.float32),
        scratch_types=[
            pltpu.VMEM((per_w,), jnp.int32),
            pltpu.VMEM((_NG, _CHUNK, d_model), jnp.float32),
            pltpu.VMEM((_NO, _CHUNK, d_model), jnp.float32),
        ]
        + [pltpu.SemaphoreType.DMA] * (_NG + _NO),
        compiler_params=pltpu.CompilerParams(use_tc_tiling_on_sc=False),
    )
    def lookup(table_hbm, idx_hbm, out_hbm, idx_v, gbuf, obuf, *sems):
        gsems = sems[:_NG]
        ssems = sems[_NG:]
        wid = lax.axis_index("s") * _NUM_CORES + lax.axis_index("c")
        pltpu.sync_copy(idx_hbm.at[pl.ds(wid * per_w, per_w)], idx_v)

        def gather_desc(j, b):
            return pltpu.make_async_copy(
                table_hbm.at[idx_v.at[pl.ds(j * _CHUNK, _CHUNK)]],
                gbuf.at[b],
                gsems[b],
            )

        def scatter_desc(j, bs):
            base = (wid * n_chunk + j) * _CHUNK
            return pltpu.make_async_copy(
                obuf.at[bs], out_hbm.at[pl.ds(base, _CHUNK)], ssems[bs]
            )

        for b in range(_NG):
            gather_desc(jnp.int32(b), b).start()

        def outer(g, carry):
            for b in range(_NG):
                j = g * _NG + b
                gather_desc(j, b).wait()
                bs = b % _NO
                if b < _NO:

                    @pl.when(g > 0)
                    def _():
                        scatter_desc(j, bs).wait()

                else:
                    scatter_desc(j, bs).wait()

                def row_body(i, c, b=b, bs=bs):
                    for k in range(4):
                        r = i * 4 + k
                        for t in range(n_vec):
                            sl = pl.ds(t * _LANES, _LANES)
                            obuf[bs, r, sl] = gbuf[b, r, sl] * scale
                    return c

                lax.fori_loop(0, _CHUNK // 4, row_body, 0)
                scatter_desc(j, bs).start()

                @pl.when(g < n_outer - 1)
                def _(b=b, j=j):
                    gather_desc(j + _NG, b).start()

            return carry

        lax.fori_loop(0, n_outer, outer, 0)
        for bs in range(_NO):
            scatter_desc(jnp.int32(n_chunk - _NO + bs), bs).wait()

    return lookup


def kernel(X, table):
    b, s = X.shape
    _, d_model = table.shape
    n_idx = b * s
    scale = math.sqrt(d_model)
    idx = X.reshape(n_idx).astype(jnp.int32)
    out = _make_lookup(d_model, n_idx, scale)(table, idx)
    return out.reshape(b, s, d_model)


# speedup vs baseline: 1.1565x; 1.1565x over previous
import functools
import math

import jax
import jax.numpy as jnp
from jax import lax
from jax.experimental import pallas as pl
from jax.experimental.pallas import tpu as pltpu
from jax.experimental.pallas import tpu_sc as plsc

_NUM_CORES = 2
_NUM_SUBCORES = 16
_NW = _NUM_CORES * _NUM_SUBCORES
_CHUNK = 128
_NG = 4
_PAD_D = 128
_CONV_BLK = 2048


def _conv_table(table_t):
    d_model, vocab = table_t.shape
    grid = (vocab + _CONV_BLK - 1) // _CONV_BLK

    def body(x_ref, o_ref):
        o_ref[:, :d_model] = jnp.transpose(x_ref[...])

    return pl.pallas_call(
        body,
        grid=(grid,),
        in_specs=[pl.BlockSpec((d_model, _CONV_BLK), lambda i: (0, i))],
        out_specs=pl.BlockSpec((_CONV_BLK, _PAD_D), lambda i: (i, 0)),
        out_shape=jax.ShapeDtypeStruct((vocab, _PAD_D), jnp.float32),
    )(table_t)


def _transpose_out(gathered, b, s, d_model, scale):
    def body(x_ref, o_ref):
        o_ref[...] = (jnp.transpose(x_ref[:, :d_model]) * scale)[None]

    return pl.pallas_call(
        body,
        grid=(s,),
        in_specs=[pl.BlockSpec((b, _PAD_D), lambda i: (i, 0))],
        out_specs=pl.BlockSpec((1, d_model, b), lambda i: (i, 0, 0)),
        out_shape=jax.ShapeDtypeStruct((s, d_model, b), jnp.float32),
    )(gathered)


def _make_lookup(n_idx):
    per_w = n_idx // _NW
    n_chunk = per_w // _CHUNK
    mesh = plsc.VectorSubcoreMesh(core_axis_name="c", subcore_axis_name="s")

    @functools.partial(
        pl.kernel,
        mesh=mesh,
        out_type=jax.ShapeDtypeStruct((n_idx, _PAD_D), jnp.float32),
        scratch_types=[
            pltpu.VMEM((per_w,), jnp.int32),
            pltpu.VMEM((_NG, _CHUNK, _PAD_D), jnp.float32),
            pltpu.SemaphoreType.DMA,
            pltpu.SemaphoreType.DMA,
        ],
        compiler_params=pltpu.CompilerParams(use_tc_tiling_on_sc=False),
    )
    def lookup(table_hbm, idx_hbm, out_hbm, idx_v, gbuf, gsem, ssem):
        n_group = n_chunk // _NG
        wid = lax.axis_index("s") * _NUM_CORES + lax.axis_index("c")
        pltpu.sync_copy(idx_hbm.at[pl.ds(wid * per_w, per_w)], idx_v)

        def gather_desc(j, b):
            return pltpu.make_async_copy(
                table_hbm.at[idx_v.at[pl.ds(j * _CHUNK, _CHUNK)]],
                gbuf.at[b],
                gsem,
            )

        def scatter_desc(j, b):
            base = (wid * n_chunk + j) * _CHUNK
            return pltpu.make_async_copy(
                gbuf.at[b], out_hbm.at[pl.ds(base, _CHUNK)], ssem
            )

        def group(g, carry):
            j0 = g * _NG

            @pl.when(g > 0)
            def _():
                for b in range(_NG):
                    scatter_desc(j0 - _NG + b, b).wait()

            for b in range(_NG):
                gather_desc(j0 + b, b).start()
            for b in range(_NG):
                gather_desc(j0 + b, b).wait()
            for b in range(_NG):
                scatter_desc(j0 + b, b).start()
            return carry

        lax.fori_loop(0, n_group, group, 0)
        for b in range(_NG):
            scatter_desc(n_chunk - _NG + b, b).wait()

    return lookup


def kernel(X, table):
    b, s = X.shape
    _, d_model = table.shape
    n_idx = b * s
    scale = math.sqrt(d_model)
    table_p = _conv_table(jnp.transpose(table))
    idx = jnp.transpose(X).reshape(n_idx).astype(jnp.int32)
    gathered = _make_lookup(n_idx)(table_p, idx)
    out = _transpose_out(gathered, b, s, d_model, scale)
    return jnp.transpose(out, (2, 0, 1))
